# Initial kernel scaffold; baseline (speedup 1.0000x reference)
import jax
import jax.numpy as jnp
from jax.experimental import pallas as pl


def kernel(userIdx, itemIdx, L_row, L_col, L_val, L3_row, L3_col, L3_val, uEmbd, iEmbd, W_lin, b_lin, W_lin1, b_lin1, W_inter, b_inter, W_inter1, b_inter1, W1, b1, W2, b2, W3, b3):
    raise NotImplementedError("write your pallas kernel here")


# speedup vs baseline: 5.2969x; 5.2969x over previous
import functools

import jax
import jax.numpy as jnp
from jax import lax
from jax.experimental import pallas as pl
from jax.experimental.pallas import tpu as pltpu
from jax.experimental.pallas import tpu_sc as plsc

USER_N = 25000
NN = 50000
EDGES = 800000
DD = 64
BATCH = 16384

NC = 2
NS = 16
HALF = DD // 2

CH = 128
TOT_E = 2 * EDGES
NCHUNK = TOT_E // CH
SPAD = 50048
ROWS_PER_SUB = SPAD // NS
ZROWS = 136


def _sc_spmm_body(row_hbm, col_hbm, val_hbm, t4_hbm, sout_hbm,
                  row_v, col_v, val_v, gidx_v, rows_v, zbuf, acc, sem):
    c = lax.axis_index("c")
    s = lax.axis_index("s")

    def _zrow(i, carry):
        zbuf[i, pl.ds(0, 16)] = jnp.zeros((16,), jnp.float32)
        zbuf[i, pl.ds(16, 16)] = jnp.zeros((16,), jnp.float32)
        return carry

    lax.fori_loop(0, ZROWS, _zrow, 0)

    def _zcopy(i, carry):
        pltpu.sync_copy(zbuf, acc.at[pl.ds(s * ROWS_PER_SUB + i * ZROWS, ZROWS)])
        return carry

    lax.fori_loop(0, ROWS_PER_SUB // ZROWS, _zcopy, 0)
    plsc.subcore_barrier()

    def _chunk(i, carry):
        k = s + i * NS
        base = k * CH
        pltpu.sync_copy(row_hbm.at[pl.ds(base, CH)], row_v)
        pltpu.sync_copy(col_hbm.at[pl.ds(base, CH)], col_v)
        pltpu.sync_copy(val_hbm.at[pl.ds(base, CH)], val_v)

        def _gi(j, cc):
            cv = col_v[pl.ds(j * 16, 16)]
            gidx_v[pl.ds(j * 16, 16)] = cv * 2 + c
            return cc

        lax.fori_loop(0, CH // 16, _gi, 0, unroll=True)

        pltpu.async_copy(t4_hbm.at[gidx_v], rows_v, sem).wait()

        def _scale(g, cc):
            vv = val_v[pl.ds(g * 16, 16)]
            for e2 in range(16):
                bv = lax.gather(
                    vv, jnp.full((16, 1), e2, jnp.int32),
                    lax.GatherDimensionNumbers(offset_dims=(),
                                               collapsed_slice_dims=(0,),
                                               start_index_map=(0,)),
                    (1,), mode=lax.GatherScatterMode.PROMISE_IN_BOUNDS)
                e = g * 16 + e2
                r0 = rows_v[e, pl.ds(0, 16)]
                rows_v[e, pl.ds(0, 16)] = r0 * bv
                r1 = rows_v[e, pl.ds(16, 16)]
                rows_v[e, pl.ds(16, 16)] = r1 * bv
            return cc

        lax.fori_loop(0, CH // 16, _scale, 0)

        pltpu.sync_copy(rows_v, acc.at[row_v], add=True)
        return carry

    nk = (NCHUNK - s + NS - 1) // NS
    lax.fori_loop(0, nk, _chunk, 0)
    plsc.subcore_barrier()

    r0 = s * ROWS_PER_SUB
    pltpu.sync_copy(acc.at[pl.ds(r0, ROWS_PER_SUB)],
                    sout_hbm.at[c, pl.ds(r0, ROWS_PER_SUB)])


_sc_spmm = pl.kernel(
    _sc_spmm_body,
    out_type=jax.ShapeDtypeStruct((NC, SPAD, HALF), jnp.float32),
    mesh=plsc.VectorSubcoreMesh(core_axis_name="c", subcore_axis_name="s"),
    scratch_types=[
        pltpu.VMEM((CH,), jnp.int32),
        pltpu.VMEM((CH,), jnp.int32),
        pltpu.VMEM((CH,), jnp.float32),
        pltpu.VMEM((CH,), jnp.int32),
        pltpu.VMEM((CH, HALF), jnp.float32),
        pltpu.VMEM((ZROWS, HALF), jnp.float32),
        pltpu.VMEM_SHARED((SPAD, HALF), jnp.float32),
        pltpu.SemaphoreType.DMA,
    ],
    compiler_params=pltpu.CompilerParams(use_tc_tiling_on_sc=False),
)


ROWS_PER_W = BATCH // (NC * NS)
GCH = 128


def _sc_gather_body(fe_hbm, uidx_hbm, iidx_hbm, ue_hbm, ie_hbm,
                    idx_v, gbuf, sem):
    c = lax.axis_index("c")
    s = lax.axis_index("s")
    wid = s * NC + c

    def _table(idx_hbm, out_hbm):
        def _ch(j, carry):
            base = wid * ROWS_PER_W + j * GCH
            pltpu.sync_copy(idx_hbm.at[pl.ds(base, GCH)], idx_v)
            pltpu.async_copy(fe_hbm.at[idx_v], gbuf, sem).wait()
            pltpu.sync_copy(gbuf, out_hbm.at[pl.ds(base, GCH)])
            return carry

        lax.fori_loop(0, ROWS_PER_W // GCH, _ch, 0)

    _table(uidx_hbm, ue_hbm)
    _table(iidx_hbm, ie_hbm)


_sc_gather = pl.kernel(
    _sc_gather_body,
    out_type=[
        jax.ShapeDtypeStruct((BATCH, 2 * DD), jnp.float32),
        jax.ShapeDtypeStruct((BATCH, 2 * DD), jnp.float32),
    ],
    mesh=plsc.VectorSubcoreMesh(core_axis_name="c", subcore_axis_name="s"),
    scratch_types=[
        pltpu.VMEM((GCH,), jnp.int32),
        pltpu.VMEM((GCH, 2 * DD), jnp.float32),
        pltpu.SemaphoreType.DMA,
    ],
)


RB = 400
NB = NN // RB


def _tc_pre_body(f_ref, wl_ref, wi_ref, t_ref):
    x = f_ref[...]
    x2 = x * x
    wl = wl_ref[0]
    wi = wi_ref[0]
    t_ref[...] = (jnp.dot(x, wl, preferred_element_type=jnp.float32)
                  + jnp.dot(x2, wi, preferred_element_type=jnp.float32))


_tc_pre = pl.pallas_call(
    _tc_pre_body,
    grid=(2 * NB,),
    in_specs=[
        pl.BlockSpec((RB, DD), lambda i: (i % NB, 0)),
        pl.BlockSpec((1, DD, DD), lambda i: (i // NB, 0, 0)),
        pl.BlockSpec((1, DD, DD), lambda i: (i // NB, 0, 0)),
    ],
    out_specs=pl.BlockSpec((RB, DD), lambda i: (i, 0)),
    out_shape=jax.ShapeDtypeStruct((2 * NN, DD), jnp.float32),
)


def _tc_fin_body(f_ref, s0_ref, s1_ref, b_ref, out_ref):
    f = f_ref[...]
    sm = jnp.concatenate([s0_ref[0], s1_ref[0]], axis=1) + b_ref[...]
    out_ref[...] = jnp.concatenate([f, jnp.maximum(sm, 0.0)], axis=1)


_tc_fin = pl.pallas_call(
    _tc_fin_body,
    grid=(NB,),
    in_specs=[
        pl.BlockSpec((RB, DD), lambda i: (i, 0)),
        pl.BlockSpec((1, RB, HALF), lambda i: (0, i, 0)),
        pl.BlockSpec((1, RB, HALF), lambda i: (1, i, 0)),
        pl.BlockSpec((1, DD), lambda i: (0, 0)),
    ],
    out_specs=pl.BlockSpec((RB, 2 * DD), lambda i: (i, 0)),
    out_shape=jax.ShapeDtypeStruct((NN, 2 * DD), jnp.float32),
)


HB = 512
HNB = BATCH // HB


def _tc_head_body(u_ref, i_ref, w1u_ref, w1i_ref, b1_ref, w2_ref, b2_ref,
                  w3_ref, b3_ref, out_ref):
    u = u_ref[...]
    it = i_ref[...]
    h = (jnp.dot(u, w1u_ref[...], preferred_element_type=jnp.float32)
         + jnp.dot(it, w1i_ref[...], preferred_element_type=jnp.float32)
         + b1_ref[...])
    h = jnp.maximum(h, 0.0)
    h2 = jnp.dot(h, w2_ref[...], preferred_element_type=jnp.float32) + b2_ref[...]
    p = jnp.sum(h2 * w3_ref[...], axis=1, keepdims=True) + b3_ref[...]
    out_ref[...] = p


_tc_head = pl.pallas_call(
    _tc_head_body,
    grid=(HNB,),
    in_specs=[
        pl.BlockSpec((HB, 2 * DD), lambda i: (i, 0)),
        pl.BlockSpec((HB, 2 * DD), lambda i: (i, 0)),
        pl.BlockSpec((2 * DD, DD), lambda i: (0, 0)),
        pl.BlockSpec((2 * DD, DD), lambda i: (0, 0)),
        pl.BlockSpec((1, DD), lambda i: (0, 0)),
        pl.BlockSpec((DD, HALF), lambda i: (0, 0)),
        pl.BlockSpec((1, HALF), lambda i: (0, 0)),
        pl.BlockSpec((1, HALF), lambda i: (0, 0)),
        pl.BlockSpec((1, 1), lambda i: (0, 0)),
    ],
    out_specs=pl.BlockSpec((HB, 1), lambda i: (i, 0)),
    out_shape=jax.ShapeDtypeStruct((BATCH, 1), jnp.float32),
)


@jax.jit
def kernel(userIdx, itemIdx, L_row, L_col, L_val, L3_row, L3_col, L3_val,
           uEmbd, iEmbd, W_lin, b_lin, W_lin1, b_lin1, W_inter, b_inter,
           W_inter1, b_inter1, W1, b1, W2, b2, W3, b3):
    uidx = userIdx.astype(jnp.int32)
    iidx = (itemIdx + USER_N).astype(jnp.int32)

    F = jnp.concatenate([uEmbd, iEmbd], axis=0)
    cat_row = jnp.concatenate([L_row, L3_row]).astype(jnp.int32)
    cat_col = jnp.concatenate([L_col, L3_col + NN]).astype(jnp.int32)
    cat_val = jnp.concatenate([L_val, L3_val])

    wl_pair = jnp.stack([W_lin, W_lin1])
    wi_pair = jnp.stack([W_inter, W_inter1])

    t = _tc_pre(F, wl_pair, wi_pair)
    t4 = t.reshape(4 * NN, HALF)

    spair = _sc_spmm(cat_row, cat_col, cat_val, t4)

    bsum = (b_lin + b_inter + b_lin1 + b_inter1).reshape(1, DD)
    final_embd = _tc_fin(F, spair, spair, bsum)

    u_embd, i_embd = _sc_gather(final_embd, uidx, iidx)

    pred = _tc_head(u_embd, i_embd, W1[:2 * DD], W1[2 * DD:],
                    b1.reshape(1, DD), W2, b2.reshape(1, HALF),
                    W3.reshape(1, HALF), b3.reshape(1, 1))
    return (pred.reshape(BATCH), u_embd, i_embd, final_embd)


# speedup vs baseline: 12.2170x; 2.3064x over previous
import functools

import jax
import jax.numpy as jnp
from jax import lax
from jax.experimental import pallas as pl
from jax.experimental.pallas import tpu as pltpu
from jax.experimental.pallas import tpu_sc as plsc

USER_N = 25000
NN = 50000
EDGES = 800000
DD = 64
BATCH = 16384

NC = 2
NS = 16
HALF = DD // 2

CH = 128
TOT_E = 2 * EDGES
CPS = 16
NSC = 49
NK = NSC * CPS
PADE = NS * NK * CH
SPAD = 50048
ROWS_PER_SUB = SPAD // NS
ZROWS = 136


def _lane_bcast(vv, e2):
    return lax.gather(
        vv, jnp.full((16, 1), e2, jnp.int32),
        lax.GatherDimensionNumbers(offset_dims=(),
                                   collapsed_slice_dims=(0,),
                                   start_index_map=(0,)),
        (1,), mode=lax.GatherScatterMode.PROMISE_IN_BOUNDS)


def _sc_spmm_body(row_hbm, col_hbm, val_hbm, t4_hbm, sout_hbm,
                  srow, scol, sval, gidx, rbuf, zbuf, acc, gsem):
    c = lax.axis_index("c")
    s = lax.axis_index("s")

    def _zrow(i, carry):
        zbuf[i, pl.ds(0, 16)] = jnp.zeros((16,), jnp.float32)
        zbuf[i, pl.ds(16, 16)] = jnp.zeros((16,), jnp.float32)
        return carry

    lax.fori_loop(0, ZROWS, _zrow, 0)

    def _zcopy(i, carry):
        pltpu.sync_copy(zbuf, acc.at[pl.ds(s * ROWS_PER_SUB + i * ZROWS, ZROWS)])
        return carry

    lax.fori_loop(0, ROWS_PER_SUB // ZROWS, _zcopy, 0)
    plsc.subcore_barrier()

    crow0 = s * NK

    def _load_sc(sc):
        r0 = crow0 + sc * CPS
        pltpu.sync_copy(row_hbm.at[pl.ds(r0, CPS)], srow)
        pltpu.sync_copy(col_hbm.at[pl.ds(r0, CPS)], scol)
        pltpu.sync_copy(val_hbm.at[pl.ds(r0, CPS)], sval)

    def _prep_and_fire(kn):
        j = lax.rem(kn, CPS)
        b = lax.rem(kn, 2)

        def _g(g, cc):
            cv = scol[j, pl.ds(g * 16, 16)]
            gidx[b, pl.ds(g * 16, 16)] = cv * 2 + c
            return cc

        lax.fori_loop(0, CH // 16, _g, 0, unroll=True)
        pltpu.async_copy(t4_hbm.at[gidx.at[b]], rbuf.at[b], gsem.at[b])

    _load_sc(0)
    _prep_and_fire(0)

    def _iter(k, carry):
        b = lax.rem(k, 2)
        j = lax.rem(k, CPS)
        kn = k + 1
        jn = lax.rem(kn, CPS)

        @pl.when(jnp.logical_and(kn < NK, jn != 0))
        def _fire_ahead():
            _prep_and_fire(kn)

        pltpu.make_async_copy(t4_hbm.at[gidx.at[b]], rbuf.at[b], gsem.at[b]).wait()

        def _scale(g, cc):
            vv = sval[j, pl.ds(g * 16, 16)]
            for e2 in range(16):
                bv = _lane_bcast(vv, e2)
                e = g * 16 + e2
                r0 = rbuf[b, e, pl.ds(0, 16)]
                rbuf[b, e, pl.ds(0, 16)] = r0 * bv
                r1 = rbuf[b, e, pl.ds(16, 16)]
                rbuf[b, e, pl.ds(16, 16)] = r1 * bv
            return cc

        lax.fori_loop(0, CH // 16, _scale, 0)

        pltpu.sync_copy(rbuf.at[b], acc.at[srow.at[j]], add=True)

        @pl.when(jnp.logical_and(kn < NK, jn == 0))
        def _boundary():
            _load_sc(lax.div(kn, CPS))
            _prep_and_fire(kn)

        return carry

    lax.fori_loop(0, NK, _iter, 0)
    plsc.subcore_barrier()

    r0 = s * ROWS_PER_SUB
    pltpu.sync_copy(acc.at[pl.ds(r0, ROWS_PER_SUB)],
                    sout_hbm.at[c, pl.ds(r0, ROWS_PER_SUB)])


_sc_spmm = pl.kernel(
    _sc_spmm_body,
    out_type=jax.ShapeDtypeStruct((NC, SPAD, HALF), jnp.float32),
    mesh=plsc.VectorSubcoreMesh(core_axis_name="c", subcore_axis_name="s"),
    scratch_types=[
        pltpu.VMEM((CPS, CH), jnp.int32),
        pltpu.VMEM((CPS, CH), jnp.int32),
        pltpu.VMEM((CPS, CH), jnp.float32),
        pltpu.VMEM((2, CH), jnp.int32),
        pltpu.VMEM((2, CH, HALF), jnp.float32),
        pltpu.VMEM((ZROWS, HALF), jnp.float32),
        pltpu.VMEM_SHARED((SPAD, HALF), jnp.float32),
        pltpu.SemaphoreType.DMA((2,)),
    ],
    compiler_params=pltpu.CompilerParams(use_tc_tiling_on_sc=False),
)


ROWS_PER_W = BATCH // (NC * NS)
GCH = 128


def _sc_gather_body(fe_hbm, uidx_hbm, iidx_hbm, ue_hbm, ie_hbm,
                    idx_v, gbuf, sem):
    c = lax.axis_index("c")
    s = lax.axis_index("s")
    wid = s * NC + c

    def _table(idx_hbm, out_hbm):
        def _ch(j, carry):
            base = wid * ROWS_PER_W + j * GCH
            pltpu.sync_copy(idx_hbm.at[pl.ds(base, GCH)], idx_v)
            pltpu.async_copy(fe_hbm.at[idx_v], gbuf, sem).wait()
            pltpu.sync_copy(gbuf, out_hbm.at[pl.ds(base, GCH)])
            return carry

        lax.fori_loop(0, ROWS_PER_W // GCH, _ch, 0)

    _table(uidx_hbm, ue_hbm)
    _table(iidx_hbm, ie_hbm)


_sc_gather = pl.kernel(
    _sc_gather_body,
    out_type=[
        jax.ShapeDtypeStruct((BATCH, 2 * DD), jnp.float32),
        jax.ShapeDtypeStruct((BATCH, 2 * DD), jnp.float32),
    ],
    mesh=plsc.VectorSubcoreMesh(core_axis_name="c", subcore_axis_name="s"),
    scratch_types=[
        pltpu.VMEM((GCH,), jnp.int32),
        pltpu.VMEM((GCH, 2 * DD), jnp.float32),
        pltpu.SemaphoreType.DMA,
    ],
)


RB = 400
NB = NN // RB


def _tc_pre_body(f_ref, wl_ref, wi_ref, t_ref):
    x = f_ref[...]
    x2 = x * x
    wl = wl_ref[0]
    wi = wi_ref[0]
    t_ref[...] = (jnp.dot(x, wl, preferred_element_type=jnp.float32)
                  + jnp.dot(x2, wi, preferred_element_type=jnp.float32))


_tc_pre = pl.pallas_call(
    _tc_pre_body,
    grid=(2 * NB,),
    in_specs=[
        pl.BlockSpec((RB, DD), lambda i: (i % NB, 0)),
        pl.BlockSpec((1, DD, DD), lambda i: (i // NB, 0, 0)),
        pl.BlockSpec((1, DD, DD), lambda i: (i // NB, 0, 0)),
    ],
    out_specs=pl.BlockSpec((RB, DD), lambda i: (i, 0)),
    out_shape=jax.ShapeDtypeStruct((2 * NN, DD), jnp.float32),
)


def _tc_fin_body(f_ref, s0_ref, s1_ref, b_ref, out_ref):
    f = f_ref[...]
    sm = jnp.concatenate([s0_ref[0], s1_ref[0]], axis=1) + b_ref[...]
    out_ref[...] = jnp.concatenate([f, jnp.maximum(sm, 0.0)], axis=1)


_tc_fin = pl.pallas_call(
    _tc_fin_body,
    grid=(NB,),
    in_specs=[
        pl.BlockSpec((RB, DD), lambda i: (i, 0)),
        pl.BlockSpec((1, RB, HALF), lambda i: (0, i, 0)),
        pl.BlockSpec((1, RB, HALF), lambda i: (1, i, 0)),
        pl.BlockSpec((1, DD), lambda i: (0, 0)),
    ],
    out_specs=pl.BlockSpec((RB, 2 * DD), lambda i: (i, 0)),
    out_shape=jax.ShapeDtypeStruct((NN, 2 * DD), jnp.float32),
)


HB = 512
HNB = BATCH // HB


def _tc_head_body(u_ref, i_ref, w1u_ref, w1i_ref, b1_ref, w2_ref, b2_ref,
                  w3_ref, b3_ref, out_ref):
    u = u_ref[...]
    it = i_ref[...]
    h = (jnp.dot(u, w1u_ref[...], preferred_element_type=jnp.float32)
         + jnp.dot(it, w1i_ref[...], preferred_element_type=jnp.float32)
         + b1_ref[...])
    h = jnp.maximum(h, 0.0)
    h2 = jnp.dot(h, w2_ref[...], preferred_element_type=jnp.float32) + b2_ref[...]
    p = jnp.sum(h2 * w3_ref[...], axis=1, keepdims=True) + b3_ref[...]
    out_ref[...] = p


_tc_head = pl.pallas_call(
    _tc_head_body,
    grid=(HNB,),
    in_specs=[
        pl.BlockSpec((HB, 2 * DD), lambda i: (i, 0)),
        pl.BlockSpec((HB, 2 * DD), lambda i: (i, 0)),
        pl.BlockSpec((2 * DD, DD), lambda i: (0, 0)),
        pl.BlockSpec((2 * DD, DD), lambda i: (0, 0)),
        pl.BlockSpec((1, DD), lambda i: (0, 0)),
        pl.BlockSpec((DD, HALF), lambda i: (0, 0)),
        pl.BlockSpec((1, HALF), lambda i: (0, 0)),
        pl.BlockSpec((1, HALF), lambda i: (0, 0)),
        pl.BlockSpec((1, 1), lambda i: (0, 0)),
    ],
    out_specs=pl.BlockSpec((HB, 1), lambda i: (i, 0)),
    out_shape=jax.ShapeDtypeStruct((BATCH, 1), jnp.float32),
)


@jax.jit
def kernel(userIdx, itemIdx, L_row, L_col, L_val, L3_row, L3_col, L3_val,
           uEmbd, iEmbd, W_lin, b_lin, W_lin1, b_lin1, W_inter, b_inter,
           W_inter1, b_inter1, W1, b1, W2, b2, W3, b3):
    uidx = userIdx.astype(jnp.int32)
    iidx = (itemIdx + USER_N).astype(jnp.int32)

    F = jnp.concatenate([uEmbd, iEmbd], axis=0)
    npad = PADE - TOT_E
    pidx = jnp.arange(npad, dtype=jnp.int32) * 7 % NN
    cat_row = jnp.concatenate(
        [L_row.astype(jnp.int32), L3_row.astype(jnp.int32), pidx]).reshape(-1, CH)
    cat_col = jnp.concatenate(
        [L_col.astype(jnp.int32), L3_col.astype(jnp.int32) + NN, pidx]).reshape(-1, CH)
    cat_val = jnp.concatenate(
        [L_val, L3_val, jnp.zeros((npad,), jnp.float32)]).reshape(-1, CH)

    wl_pair = jnp.stack([W_lin, W_lin1])
    wi_pair = jnp.stack([W_inter, W_inter1])

    t = _tc_pre(F, wl_pair, wi_pair)
    t4 = t.reshape(4 * NN, HALF)

    spair = _sc_spmm(cat_row, cat_col, cat_val, t4)

    bsum = (b_lin + b_inter + b_lin1 + b_inter1).reshape(1, DD)
    final_embd = _tc_fin(F, spair, spair, bsum)

    u_embd, i_embd = _sc_gather(final_embd, uidx, iidx)

    pred = _tc_head(u_embd, i_embd, W1[:2 * DD], W1[2 * DD:],
                    b1.reshape(1, DD), W2, b2.reshape(1, HALF),
                    W3.reshape(1, HALF), b3.reshape(1, 1))
    return (pred.reshape(BATCH), u_embd, i_embd, final_embd)


# speedup vs baseline: 13.3267x; 1.0908x over previous
import functools

import jax
import jax.numpy as jnp
from jax import lax
from jax.experimental import pallas as pl
from jax.experimental.pallas import tpu as pltpu
from jax.experimental.pallas import tpu_sc as plsc

USER_N = 25000
NN = 50000
EDGES = 800000
DD = 64
BATCH = 16384

NC = 2
NS = 16
HALF = DD // 2

CH = 128
TOT_E = 2 * EDGES
CPS = 16
NSC = 49
NK = NSC * CPS
PADE = NS * NK * CH
SPAD = 50048
ROWS_PER_SUB = SPAD // NS
ZROWS = 136


def _lane_bcast(vv, e2):
    return lax.gather(
        vv, jnp.full((16, 1), e2, jnp.int32),
        lax.GatherDimensionNumbers(offset_dims=(),
                                   collapsed_slice_dims=(0,),
                                   start_index_map=(0,)),
        (1,), mode=lax.GatherScatterMode.PROMISE_IN_BOUNDS)


def _sc_spmm_body(row_hbm, col_hbm, val_hbm, t4_hbm, sout_hbm,
                  srow, scol, sval, gidx, rbuf, zbuf, acc, gsem):
    c = lax.axis_index("c")
    s = lax.axis_index("s")

    def _zrow(i, carry):
        zbuf[i, pl.ds(0, 16)] = jnp.zeros((16,), jnp.float32)
        zbuf[i, pl.ds(16, 16)] = jnp.zeros((16,), jnp.float32)
        return carry

    lax.fori_loop(0, ZROWS, _zrow, 0)

    def _zcopy(i, carry):
        pltpu.sync_copy(zbuf, acc.at[pl.ds(s * ROWS_PER_SUB + i * ZROWS, ZROWS)])
        return carry

    lax.fori_loop(0, ROWS_PER_SUB // ZROWS, _zcopy, 0)
    plsc.subcore_barrier()

    crow0 = s * NK

    def _load_sc(sc):
        r0 = crow0 + sc * CPS
        pltpu.sync_copy(row_hbm.at[pl.ds(r0, CPS)], srow)
        pltpu.sync_copy(col_hbm.at[pl.ds(r0, CPS)], scol)
        pltpu.sync_copy(val_hbm.at[pl.ds(r0, CPS)], sval)

    cbase = c * NN

    def _prep_and_fire(kn):
        j = lax.rem(kn, CPS)
        b = lax.rem(kn, 2)

        def _g(g, cc):
            cv = scol[j, pl.ds(g * 16, 16)]
            gidx[b, pl.ds(g * 16, 16)] = cv + cbase
            return cc

        lax.fori_loop(0, CH // 16, _g, 0, unroll=True)
        pltpu.async_copy(t4_hbm.at[gidx.at[b]], rbuf.at[b], gsem.at[b])

    _load_sc(0)
    _prep_and_fire(0)

    def _iter(k, carry):
        b = lax.rem(k, 2)
        j = lax.rem(k, CPS)
        kn = k + 1
        jn = lax.rem(kn, CPS)

        @pl.when(jnp.logical_and(kn < NK, jn != 0))
        def _fire_ahead():
            _prep_and_fire(kn)

        pltpu.make_async_copy(t4_hbm.at[gidx.at[b]], rbuf.at[b], gsem.at[b]).wait()

        def _scale(g, cc):
            vv = sval[j, pl.ds(g * 16, 16)]
            for e2 in range(16):
                bv = _lane_bcast(vv, e2)
                e = g * 16 + e2
                r0 = rbuf[b, e, pl.ds(0, 16)]
                rbuf[b, e, pl.ds(0, 16)] = r0 * bv
                r1 = rbuf[b, e, pl.ds(16, 16)]
                rbuf[b, e, pl.ds(16, 16)] = r1 * bv
            return cc

        lax.fori_loop(0, CH // 16, _scale, 0)

        pltpu.sync_copy(rbuf.at[b], acc.at[srow.at[j]], add=True)

        @pl.when(jnp.logical_and(kn < NK, jn == 0))
        def _boundary():
            _load_sc(lax.div(kn, CPS))
            _prep_and_fire(kn)

        return carry

    lax.fori_loop(0, NK, _iter, 0)
    plsc.subcore_barrier()

    r0 = s * ROWS_PER_SUB
    pltpu.sync_copy(acc.at[pl.ds(r0, ROWS_PER_SUB)],
                    sout_hbm.at[c, pl.ds(r0, ROWS_PER_SUB)])


_sc_spmm = pl.kernel(
    _sc_spmm_body,
    out_type=jax.ShapeDtypeStruct((NC, SPAD, HALF), jnp.float32),
    mesh=plsc.VectorSubcoreMesh(core_axis_name="c", subcore_axis_name="s"),
    scratch_types=[
        pltpu.VMEM((CPS, CH), jnp.int32),
        pltpu.VMEM((CPS, CH), jnp.int32),
        pltpu.VMEM((CPS, CH), jnp.float32),
        pltpu.VMEM((2, CH), jnp.int32),
        pltpu.VMEM((2, CH, HALF), jnp.float32),
        pltpu.VMEM((ZROWS, HALF), jnp.float32),
        pltpu.VMEM_SHARED((SPAD, HALF), jnp.float32),
        pltpu.SemaphoreType.DMA((2,)),
    ],
    compiler_params=pltpu.CompilerParams(use_tc_tiling_on_sc=False),
)


ROWS_PER_W = BATCH // (NC * NS)
GCH = 128


def _sc_gather_body(fe_hbm, uidx_hbm, iidx_hbm, ue_hbm, ie_hbm,
                    idx_v, gbuf, sem):
    c = lax.axis_index("c")
    s = lax.axis_index("s")
    wid = s * NC + c

    def _table(idx_hbm, out_hbm):
        def _ch(j, carry):
            base = wid * ROWS_PER_W + j * GCH
            pltpu.sync_copy(idx_hbm.at[pl.ds(base, GCH)], idx_v)
            pltpu.async_copy(fe_hbm.at[idx_v], gbuf, sem).wait()
            pltpu.sync_copy(gbuf, out_hbm.at[pl.ds(base, GCH)])
            return carry

        lax.fori_loop(0, ROWS_PER_W // GCH, _ch, 0)

    _table(uidx_hbm, ue_hbm)
    _table(iidx_hbm, ie_hbm)


_sc_gather = pl.kernel(
    _sc_gather_body,
    out_type=[
        jax.ShapeDtypeStruct((BATCH, 2 * DD), jnp.float32),
        jax.ShapeDtypeStruct((BATCH, 2 * DD), jnp.float32),
    ],
    mesh=plsc.VectorSubcoreMesh(core_axis_name="c", subcore_axis_name="s"),
    scratch_types=[
        pltpu.VMEM((GCH,), jnp.int32),
        pltpu.VMEM((GCH, 2 * DD), jnp.float32),
        pltpu.SemaphoreType.DMA,
    ],
)


RB = 2000
NB = NN // RB


def _tc_pre_body(f_ref, wa_ref, wb_ref, t_ref):
    x = f_ref[...]
    x2 = x * x
    t_ref[...] = (jnp.dot(x, wa_ref[0], preferred_element_type=jnp.float32)
                  + jnp.dot(x2, wb_ref[0], preferred_element_type=jnp.float32))


_tc_pre = pl.pallas_call(
    _tc_pre_body,
    grid=(4 * NB,),
    in_specs=[
        pl.BlockSpec((RB, DD), lambda i: (i % NB, 0)),
        pl.BlockSpec((1, DD, HALF), lambda i: (i // NB, 0, 0)),
        pl.BlockSpec((1, DD, HALF), lambda i: (i // NB, 0, 0)),
    ],
    out_specs=pl.BlockSpec((RB, HALF), lambda i: (i, 0)),
    out_shape=jax.ShapeDtypeStruct((4 * NN, HALF), jnp.float32),
)


def _tc_fin_body(f_ref, s0_ref, s1_ref, b_ref, out_ref):
    f = f_ref[...]
    sm = jnp.concatenate([s0_ref[0], s1_ref[0]], axis=1) + b_ref[...]
    out_ref[...] = jnp.concatenate([f, jnp.maximum(sm, 0.0)], axis=1)


FRB = 2000
FNB = NN // FRB

_tc_fin = pl.pallas_call(
    _tc_fin_body,
    grid=(FNB,),
    in_specs=[
        pl.BlockSpec((FRB, DD), lambda i: (i, 0)),
        pl.BlockSpec((1, FRB, HALF), lambda i: (0, i, 0)),
        pl.BlockSpec((1, FRB, HALF), lambda i: (1, i, 0)),
        pl.BlockSpec((1, DD), lambda i: (0, 0)),
    ],
    out_specs=pl.BlockSpec((FRB, 2 * DD), lambda i: (i, 0)),
    out_shape=jax.ShapeDtypeStruct((NN, 2 * DD), jnp.float32),
)


HB = 2048
HNB = BATCH // HB


def _tc_head_body(u_ref, i_ref, w1u_ref, w1i_ref, b1_ref, w2_ref, b2_ref,
                  w3_ref, b3_ref, out_ref):
    u = u_ref[...]
    it = i_ref[...]
    h = (jnp.dot(u, w1u_ref[...], preferred_element_type=jnp.float32)
         + jnp.dot(it, w1i_ref[...], preferred_element_type=jnp.float32)
         + b1_ref[...])
    h = jnp.maximum(h, 0.0)
    h2 = jnp.dot(h, w2_ref[...], preferred_element_type=jnp.float32) + b2_ref[...]
    p = jnp.sum(h2 * w3_ref[...], axis=1, keepdims=True) + b3_ref[...]
    out_ref[...] = p


_tc_head = pl.pallas_call(
    _tc_head_body,
    grid=(HNB,),
    in_specs=[
        pl.BlockSpec((HB, 2 * DD), lambda i: (i, 0)),
        pl.BlockSpec((HB, 2 * DD), lambda i: (i, 0)),
        pl.BlockSpec((2 * DD, DD), lambda i: (0, 0)),
        pl.BlockSpec((2 * DD, DD), lambda i: (0, 0)),
        pl.BlockSpec((1, DD), lambda i: (0, 0)),
        pl.BlockSpec((DD, HALF), lambda i: (0, 0)),
        pl.BlockSpec((1, HALF), lambda i: (0, 0)),
        pl.BlockSpec((1, HALF), lambda i: (0, 0)),
        pl.BlockSpec((1, 1), lambda i: (0, 0)),
    ],
    out_specs=pl.BlockSpec((HB, 1), lambda i: (i, 0)),
    out_shape=jax.ShapeDtypeStruct((BATCH, 1), jnp.float32),
)


@jax.jit
def kernel(userIdx, itemIdx, L_row, L_col, L_val, L3_row, L3_col, L3_val,
           uEmbd, iEmbd, W_lin, b_lin, W_lin1, b_lin1, W_inter, b_inter,
           W_inter1, b_inter1, W1, b1, W2, b2, W3, b3):
    uidx = userIdx.astype(jnp.int32)
    iidx = (itemIdx + USER_N).astype(jnp.int32)

    F = jnp.concatenate([uEmbd, iEmbd], axis=0)
    npad = PADE - TOT_E
    pidx = jnp.arange(npad, dtype=jnp.int32) * 7 % NN
    cat_row = jnp.concatenate(
        [L_row.astype(jnp.int32), L3_row.astype(jnp.int32), pidx]).reshape(-1, CH)
    cat_col = jnp.concatenate(
        [L_col.astype(jnp.int32), L3_col.astype(jnp.int32) + 2 * NN,
         pidx]).reshape(-1, CH)
    cat_val = jnp.concatenate(
        [L_val, L3_val, jnp.zeros((npad,), jnp.float32)]).reshape(-1, CH)

    wa = jnp.stack([W_lin[:, :HALF], W_lin[:, HALF:],
                    W_lin1[:, :HALF], W_lin1[:, HALF:]])
    wb = jnp.stack([W_inter[:, :HALF], W_inter[:, HALF:],
                    W_inter1[:, :HALF], W_inter1[:, HALF:]])

    t4 = _tc_pre(F, wa, wb)

    spair = _sc_spmm(cat_row, cat_col, cat_val, t4)

    bsum = (b_lin + b_inter + b_lin1 + b_inter1).reshape(1, DD)
    final_embd = _tc_fin(F, spair, spair, bsum)

    u_embd, i_embd = _sc_gather(final_embd, uidx, iidx)

    pred = _tc_head(u_embd, i_embd, W1[:2 * DD], W1[2 * DD:],
                    b1.reshape(1, DD), W2, b2.reshape(1, HALF),
                    W3.reshape(1, HALF), b3.reshape(1, 1))
    return (pred.reshape(BATCH), u_embd, i_embd, final_embd)
